# baseline (speedup 1.0000x reference)
import functools

import jax
import jax.numpy as jnp
from jax import lax
from jax.experimental import pallas as pl
from jax.experimental.pallas import tpu as pltpu
from jax.experimental.pallas import tpu_sc as plsc

_B, _L, _D = 1024, 200, 128
_K = 4
_BC = _B // _K
_NC, _NS = 2, 16
_NW = _NC * _NS
_CHUNK = 64
_NBUF = 5
_KD = 2
_ROWS_W = (_BC * _L) // _NW
_NCHUNK = _ROWS_W // _CHUNK


def _sc_gather_body(uh_hbm, nodes_hbm, utab_hbm, itab_hbm, uout_hbm, iout_hbm,
                    idx_v, bufs, isem, ibuf, gsems, wsems):
    wid = lax.axis_index("s") * _NC + lax.axis_index("c")
    base = wid * _ROWS_W

    pltpu.sync_copy(uh_hbm.at[wid], idx_v)

    @pl.when(wid < _BC // _CHUNK)
    def _():
        pltpu.sync_copy(nodes_hbm.at[wid], ibuf[0])
        pltpu.async_copy(itab_hbm.at[ibuf[0].at[0]], ibuf[1], isem).wait()
        pltpu.sync_copy(ibuf[1], iout_hbm.at[pl.ds(wid * _CHUNK, _CHUNK)])

    for b in range(_NBUF):
        pltpu.async_copy(utab_hbm.at[idx_v.at[b]], bufs[b], gsems[b])

    def chunk_iter(j0):
        for b in range(_NBUF):
            j = j0 + b
            pltpu.make_async_copy(utab_hbm.at[idx_v.at[b]], bufs[b],
                                  gsems[b]).wait()
            pltpu.async_copy(
                bufs[b], uout_hbm.at[pl.ds(base + j * _CHUNK, _CHUNK)],
                wsems[b])

            bp = (b - _KD) % _NBUF
            jp = j - _KD

            @pl.when(jnp.logical_and(jp >= 0,
                                     jp + _NBUF <= _NCHUNK - 1))
            def _():
                pltpu.make_async_copy(
                    bufs[bp], uout_hbm.at[pl.ds(base + jp * _CHUNK, _CHUNK)],
                    wsems[bp]).wait()
                pltpu.async_copy(utab_hbm.at[idx_v.at[jp + _NBUF]], bufs[bp],
                                 gsems[bp])

    lax.fori_loop(0, _NCHUNK // _NBUF, lambda i, c: (chunk_iter(i * _NBUF), c)[1],
                  0, unroll=False)

    for b in range(_NBUF):
        j = _NCHUNK - _NBUF + b
        pltpu.make_async_copy(
            bufs[b], uout_hbm.at[pl.ds(base + j * _CHUNK, _CHUNK)],
            wsems[b]).wait()


def _sc_gather(user_hist, nodes, user_table, item_table):
    mesh = plsc.VectorSubcoreMesh(core_axis_name="c", subcore_axis_name="s")
    uh3d = user_hist.reshape(_NW, _NCHUNK, _CHUNK).astype(jnp.int32)
    nodes3d = nodes.reshape(_BC // _CHUNK, 1, _CHUNK).astype(jnp.int32)
    gdt, gw = user_table.dtype, user_table.shape[1]
    scratch = [
        pltpu.VMEM((_NCHUNK, _CHUNK), jnp.int32),
        [pltpu.VMEM((_CHUNK, gw), gdt) for _ in range(_NBUF)],
        pltpu.SemaphoreType.DMA,
        [pltpu.VMEM((1, _CHUNK), jnp.int32),
         pltpu.VMEM((_CHUNK, _D), jnp.float32)],
        [pltpu.SemaphoreType.DMA for _ in range(_NBUF)],
        [pltpu.SemaphoreType.DMA for _ in range(_NBUF)],
    ]
    fn = pl.kernel(
        _sc_gather_body,
        out_type=[jax.ShapeDtypeStruct((_BC * _L, gw), gdt),
                  jax.ShapeDtypeStruct((_BC, _D), jnp.float32)],
        mesh=mesh,
        scratch_types=scratch,
    )
    return fn(uh3d, nodes3d, user_table, item_table)


def _tc_body(gath_ref, rh_ref, wi_ref, ln1_ref, ln1b_ref, ln2_ref, ln2b_ref,
             ln3_ref, ln3b_ref, att1_ref, att1b_ref, att2_ref, att2b_ref,
             rtab_ref, out_ref, *, nb):
    f32 = jnp.float32
    dot = lambda a, b: lax.dot_general(a, b, (((1,), (1,)), ((), ())),
                                       preferred_element_type=f32)

    g = gath_ref[...]
    rh = rh_ref[...]
    wi = wi_ref[...]

    ln1 = ln1_ref[...]
    rfold = dot(rtab_ref[...], ln1[:, _D:]) + ln1b_ref[...]
    rfold8 = jnp.concatenate([rfold, jnp.zeros((2, _D), f32)], axis=0)
    oh = (rh == lax.broadcasted_iota(jnp.int32, (1, 8), 1)).astype(f32)
    rpart = lax.dot_general(oh, rfold8, (((1,), (0,)), ((), ())),
                            preferred_element_type=f32)

    f = jnp.maximum(dot(g, ln1[:, :_D]) + rpart, 0.0)

    rows = nb * _L
    att1 = att1_ref[...]
    c = dot(wi, att1[:, _D:]) + att1b_ref[...]
    hmat = dot(f, att1[:, :_D]).reshape(nb, _L, _D)
    h = jnp.maximum(hmat + c[:, None, :], 0.0).reshape(rows, _D)
    st = lax.dot_general(att2_ref[...], h, (((1,), (1,)), ((), ())),
                         preferred_element_type=f32)
    ex = jnp.exp(st - jnp.max(st))
    seg = lax.broadcasted_iota(jnp.int32, (1, rows), 1) // _L
    m = jnp.where(seg == lax.broadcasted_iota(jnp.int32, (nb, 1), 0),
                  ex, 0.0)
    denom = jnp.sum(m, axis=1, keepdims=True)
    zsum = lax.dot_general(m, f, (((1,), (0,)), ((), ())),
                           preferred_element_type=f32)
    z = zsum / denom

    z2 = jnp.maximum(dot(z, ln2_ref[...]) + ln2b_ref[...], 0.0)
    ln3 = ln3_ref[...]
    out = jnp.maximum(dot(wi, ln3[:, :_D]) + dot(z2, ln3[:, _D:])
                      + ln3b_ref[...], 0.0)
    out_ref[...] = out


def _tc_compute(gathered, rh2d, w_item, rating_table,
                ln1_w, ln1_b, ln2_w, ln2_b, ln3_w, ln3_b,
                att1_w, att1_b, att2_w, att2_b, *, nb=64, interpret=False):
    bc = w_item.shape[0]
    grid = (bc // nb,)
    row1 = lambda x: x.reshape(1, -1).astype(jnp.float32)
    full = lambda a: pl.BlockSpec(a.shape, lambda i: (0,) * a.ndim)
    in_specs = [
        pl.BlockSpec((nb * _L, gathered.shape[1]), lambda i: (i, 0)),
        pl.BlockSpec((nb * _L, 1), lambda i: (i, 0)),
        pl.BlockSpec((nb, _D), lambda i: (i, 0)),
        full(ln1_w), full(row1(ln1_b)), full(ln2_w), full(row1(ln2_b)),
        full(ln3_w), full(row1(ln3_b)), full(att1_w), full(row1(att1_b)),
        full(row1(att2_w)), full(row1(att2_b)), full(rating_table),
    ]
    return pl.pallas_call(
        functools.partial(_tc_body, nb=nb),
        grid=grid,
        in_specs=in_specs,
        out_specs=pl.BlockSpec((nb, _D), lambda i: (i, 0)),
        out_shape=jax.ShapeDtypeStruct((bc, _D), jnp.float32),
        interpret=interpret,
    )(gathered, rh2d, w_item, ln1_w, row1(ln1_b), ln2_w, row1(ln2_b),
      ln3_w, row1(ln3_b), att1_w, row1(att1_b), row1(att2_w), row1(att2_b),
      rating_table)


def kernel(user_hist, rating_hist, nodes, user_table, item_table,
           rating_table, ln1_w, ln1_b, ln2_w, ln2_b, ln3_w, ln3_b,
           att1_w, att1_b, att2_w, att2_b):
    rh_all = rating_hist.reshape(_B * _L, 1).astype(jnp.int32)
    outs = []
    for k in range(_K):
        sl = slice(k * _BC, (k + 1) * _BC)
        gathered, w_item = _sc_gather(user_hist[sl], nodes[sl], user_table,
                                      item_table)
        outs.append(_tc_compute(
            gathered, rh_all[k * _BC * _L:(k + 1) * _BC * _L], w_item,
            rating_table, ln1_w, ln1_b, ln2_w, ln2_b,
            ln3_w, ln3_b, att1_w, att1_b, att2_w, att2_b))
    return jnp.concatenate(outs, axis=0)


# speedup vs baseline: 1.5694x; 1.5694x over previous
import functools

import jax
import jax.numpy as jnp
from jax import lax
from jax.experimental import pallas as pl
from jax.experimental.pallas import tpu as pltpu
from jax.experimental.pallas import tpu_sc as plsc

_B, _L, _D = 1024, 200, 128
_K = 4
_BC = _B // _K
_NC, _NS = 2, 16
_NW = _NC * _NS
_CHUNK = 64
_NBUF = 5
_KD = 2
_ROWS_W = (_BC * _L) // _NW
_NCHUNK = _ROWS_W // _CHUNK


def _sc_gather_body(uh_hbm, nodes_hbm, utab_hbm, itab_hbm, uout_hbm, iout_hbm,
                    idx_v, bufs, isem, ibuf, gsems, wsems):
    wid = lax.axis_index("s") * _NC + lax.axis_index("c")
    base = wid * _ROWS_W

    pltpu.sync_copy(uh_hbm.at[wid], idx_v)

    @pl.when(wid < _BC // _CHUNK)
    def _():
        pltpu.sync_copy(nodes_hbm.at[wid], ibuf[0])
        pltpu.async_copy(itab_hbm.at[ibuf[0].at[0]], ibuf[1], isem).wait()
        pltpu.sync_copy(ibuf[1], iout_hbm.at[pl.ds(wid * _CHUNK, _CHUNK)])

    for b in range(_NBUF):
        pltpu.async_copy(utab_hbm.at[idx_v.at[b]], bufs[b], gsems[b])

    def chunk_iter(j0):
        for b in range(_NBUF):
            j = j0 + b
            pltpu.make_async_copy(utab_hbm.at[idx_v.at[b]], bufs[b],
                                  gsems[b]).wait()
            pltpu.async_copy(
                bufs[b], uout_hbm.at[pl.ds(base + j * _CHUNK, _CHUNK)],
                wsems[b])

            bp = (b - _KD) % _NBUF
            jp = j - _KD

            @pl.when(jnp.logical_and(jp >= 0,
                                     jp + _NBUF <= _NCHUNK - 1))
            def _():
                pltpu.make_async_copy(
                    bufs[bp], uout_hbm.at[pl.ds(base + jp * _CHUNK, _CHUNK)],
                    wsems[bp]).wait()
                pltpu.async_copy(utab_hbm.at[idx_v.at[jp + _NBUF]], bufs[bp],
                                 gsems[bp])

    lax.fori_loop(0, _NCHUNK // _NBUF, lambda i, c: (chunk_iter(i * _NBUF), c)[1],
                  0, unroll=False)

    for b in range(_NBUF):
        j = _NCHUNK - _NBUF + b
        pltpu.make_async_copy(
            bufs[b], uout_hbm.at[pl.ds(base + j * _CHUNK, _CHUNK)],
            wsems[b]).wait()


def _sc_gather(user_hist, nodes, user_table, item_table):
    mesh = plsc.VectorSubcoreMesh(core_axis_name="c", subcore_axis_name="s")
    uh3d = user_hist.reshape(_NW, _NCHUNK, _CHUNK).astype(jnp.int32)
    nodes3d = nodes.reshape(_BC // _CHUNK, 1, _CHUNK).astype(jnp.int32)
    gdt, gw = user_table.dtype, user_table.shape[1]
    scratch = [
        pltpu.VMEM((_NCHUNK, _CHUNK), jnp.int32),
        [pltpu.VMEM((_CHUNK, gw), gdt) for _ in range(_NBUF)],
        pltpu.SemaphoreType.DMA,
        [pltpu.VMEM((1, _CHUNK), jnp.int32),
         pltpu.VMEM((_CHUNK, _D), jnp.float32)],
        [pltpu.SemaphoreType.DMA for _ in range(_NBUF)],
        [pltpu.SemaphoreType.DMA for _ in range(_NBUF)],
    ]
    fn = pl.kernel(
        _sc_gather_body,
        out_type=[jax.ShapeDtypeStruct((_BC * _L, gw), gdt),
                  jax.ShapeDtypeStruct((_BC, _D), jnp.float32)],
        mesh=mesh,
        scratch_types=scratch,
    )
    return fn(uh3d, nodes3d, user_table, item_table)


def _xform_body(utab_ref, ln1_ref, out_ref):
    r = lax.dot_general(utab_ref[...], ln1_ref[...][:, :_D],
                        (((1,), (1,)), ((), ())),
                        preferred_element_type=jnp.float32)
    out_ref[...] = r


def _transform_table(user_table, ln1_w, *, rb=2000, interpret=False):
    n = user_table.shape[0]
    return pl.pallas_call(
        _xform_body,
        grid=(n // rb,),
        in_specs=[pl.BlockSpec((rb, _D), lambda i: (i, 0)),
                  pl.BlockSpec(ln1_w.shape, lambda i: (0, 0))],
        out_specs=pl.BlockSpec((rb, _D), lambda i: (i, 0)),
        out_shape=jax.ShapeDtypeStruct((n, _D), jnp.float32),
        interpret=interpret,
    )(user_table, ln1_w)


def _tc_body(gath_ref, rh_ref, wi_ref, ln1_ref, ln1b_ref, ln2_ref, ln2b_ref,
             ln3_ref, ln3b_ref, att1_ref, att1b_ref, att2_ref, att2b_ref,
             rtab_ref, out_ref, *, nb):
    f32 = jnp.float32
    dot = lambda a, b: lax.dot_general(a, b, (((1,), (1,)), ((), ())),
                                       preferred_element_type=f32)

    g = gath_ref[...]
    rh = rh_ref[...]
    wi = wi_ref[...]

    ln1 = ln1_ref[...]
    rfold = dot(rtab_ref[...], ln1[:, _D:]) + ln1b_ref[...]
    rfold8 = jnp.concatenate([rfold, jnp.zeros((2, _D), f32)], axis=0)
    oht = (rh == lax.broadcasted_iota(jnp.int32, (8, 1), 0)).astype(f32)
    rpart = lax.dot_general(oht, rfold8, (((0,), (0,)), ((), ())),
                            preferred_element_type=f32)

    f = jnp.maximum(g + rpart, 0.0)

    rows = nb * _L
    att1 = att1_ref[...]
    c = dot(wi, att1[:, _D:]) + att1b_ref[...]
    hmat = dot(f, att1[:, :_D]).reshape(nb, _L, _D)
    h = jnp.maximum(hmat + c[:, None, :], 0.0).reshape(rows, _D)
    st = lax.dot_general(att2_ref[...], h, (((1,), (1,)), ((), ())),
                         preferred_element_type=f32)
    ex = jnp.exp(st - jnp.max(st))
    seg = lax.broadcasted_iota(jnp.int32, (1, rows), 1) // _L
    m = jnp.where(seg == lax.broadcasted_iota(jnp.int32, (nb, 1), 0),
                  ex, 0.0)
    denom = jnp.sum(m, axis=1, keepdims=True)
    zsum = lax.dot_general(m, f, (((1,), (0,)), ((), ())),
                           preferred_element_type=f32)
    z = zsum / denom

    z2 = jnp.maximum(dot(z, ln2_ref[...]) + ln2b_ref[...], 0.0)
    ln3 = ln3_ref[...]
    out = jnp.maximum(dot(wi, ln3[:, :_D]) + dot(z2, ln3[:, _D:])
                      + ln3b_ref[...], 0.0)
    out_ref[...] = out


def _tc_compute(gathered, rh2d, w_item, rating_table,
                ln1_w, ln1_b, ln2_w, ln2_b, ln3_w, ln3_b,
                att1_w, att1_b, att2_w, att2_b, *, nb=64, interpret=False):
    bc = w_item.shape[0]
    grid = (bc // nb,)
    row1 = lambda x: x.reshape(1, -1).astype(jnp.float32)
    full = lambda a: pl.BlockSpec(a.shape, lambda i: (0,) * a.ndim)
    in_specs = [
        pl.BlockSpec((nb * _L, gathered.shape[1]), lambda i: (i, 0)),
        pl.BlockSpec((1, nb * _L), lambda i: (0, i)),
        pl.BlockSpec((nb, _D), lambda i: (i, 0)),
        full(ln1_w), full(row1(ln1_b)), full(ln2_w), full(row1(ln2_b)),
        full(ln3_w), full(row1(ln3_b)), full(att1_w), full(row1(att1_b)),
        full(row1(att2_w)), full(row1(att2_b)), full(rating_table),
    ]
    return pl.pallas_call(
        functools.partial(_tc_body, nb=nb),
        grid=grid,
        in_specs=in_specs,
        out_specs=pl.BlockSpec((nb, _D), lambda i: (i, 0)),
        out_shape=jax.ShapeDtypeStruct((bc, _D), jnp.float32),
        interpret=interpret,
    )(gathered, rh2d, w_item, ln1_w, row1(ln1_b), ln2_w, row1(ln2_b),
      ln3_w, row1(ln3_b), att1_w, row1(att1_b), row1(att2_w), row1(att2_b),
      rating_table)


def kernel(user_hist, rating_hist, nodes, user_table, item_table,
           rating_table, ln1_w, ln1_b, ln2_w, ln2_b, ln3_w, ln3_b,
           att1_w, att1_b, att2_w, att2_b):
    ut2 = _transform_table(user_table, ln1_w)
    rh_t = rating_hist.reshape(1, _B * _L).astype(jnp.int32)
    outs = []
    for k in range(_K):
        sl = slice(k * _BC, (k + 1) * _BC)
        gathered, w_item = _sc_gather(user_hist[sl], nodes[sl], ut2,
                                      item_table)
        outs.append(_tc_compute(
            gathered, rh_t[:, k * _BC * _L:(k + 1) * _BC * _L], w_item,
            rating_table, ln1_w, ln1_b, ln2_w, ln2_b,
            ln3_w, ln3_b, att1_w, att1_b, att2_w, att2_b))
    return jnp.concatenate(outs, axis=0)


# speedup vs baseline: 1.8812x; 1.1987x over previous
import functools

import jax
import jax.numpy as jnp
from jax import lax
from jax.experimental import pallas as pl
from jax.experimental.pallas import tpu as pltpu
from jax.experimental.pallas import tpu_sc as plsc

_B, _L, _D = 1024, 200, 128
_K = 4
_BC = _B // _K
_NC, _NS = 2, 16
_NW = _NC * _NS
_CHUNK = 64
_NBUF = 5
_KD = 2
_ROWS_W = (_BC * _L) // _NW
_NCHUNK = _ROWS_W // _CHUNK


def _sc_gather_body(uh_hbm, nodes_hbm, utab_hbm, itab_hbm, uout_hbm, iout_hbm,
                    idx_v, bufs, isem, ibuf, gsems, wsems):
    wid = lax.axis_index("s") * _NC + lax.axis_index("c")
    base = wid * _ROWS_W

    pltpu.sync_copy(uh_hbm.at[wid], idx_v)

    @pl.when(wid < _BC // _CHUNK)
    def _():
        pltpu.sync_copy(nodes_hbm.at[wid], ibuf[0])
        pltpu.async_copy(itab_hbm.at[ibuf[0].at[0]], ibuf[1], isem).wait()
        pltpu.sync_copy(ibuf[1], iout_hbm.at[pl.ds(wid * _CHUNK, _CHUNK)])

    for b in range(_NBUF):
        pltpu.async_copy(utab_hbm.at[idx_v.at[b]], bufs[b], gsems[b])

    def chunk_iter(j0):
        for b in range(_NBUF):
            j = j0 + b
            pltpu.make_async_copy(utab_hbm.at[idx_v.at[b]], bufs[b],
                                  gsems[b]).wait()
            pltpu.async_copy(
                bufs[b], uout_hbm.at[pl.ds(base + j * _CHUNK, _CHUNK)],
                wsems[b])

            bp = (b - _KD) % _NBUF
            jp = j - _KD

            @pl.when(jnp.logical_and(jp >= 0,
                                     jp + _NBUF <= _NCHUNK - 1))
            def _():
                pltpu.make_async_copy(
                    bufs[bp], uout_hbm.at[pl.ds(base + jp * _CHUNK, _CHUNK)],
                    wsems[bp]).wait()
                pltpu.async_copy(utab_hbm.at[idx_v.at[jp + _NBUF]], bufs[bp],
                                 gsems[bp])

    lax.fori_loop(0, _NCHUNK // _NBUF, lambda i, c: (chunk_iter(i * _NBUF), c)[1],
                  0, unroll=False)

    for b in range(_NBUF):
        j = _NCHUNK - _NBUF + b
        pltpu.make_async_copy(
            bufs[b], uout_hbm.at[pl.ds(base + j * _CHUNK, _CHUNK)],
            wsems[b]).wait()


def _sc_gather(user_hist, nodes, user_table, item_table):
    mesh = plsc.VectorSubcoreMesh(core_axis_name="c", subcore_axis_name="s")
    uh3d = user_hist.reshape(_NW, _NCHUNK, _CHUNK).astype(jnp.int32)
    nodes3d = nodes.reshape(_BC // _CHUNK, 1, _CHUNK).astype(jnp.int32)
    gdt, gw = user_table.dtype, user_table.shape[1]
    scratch = [
        pltpu.VMEM((_NCHUNK, _CHUNK), jnp.int32),
        [pltpu.VMEM((_CHUNK, gw), gdt) for _ in range(_NBUF)],
        pltpu.SemaphoreType.DMA,
        [pltpu.VMEM((1, _CHUNK), jnp.int32),
         pltpu.VMEM((_CHUNK, _D), jnp.float32)],
        [pltpu.SemaphoreType.DMA for _ in range(_NBUF)],
        [pltpu.SemaphoreType.DMA for _ in range(_NBUF)],
    ]
    fn = pl.kernel(
        _sc_gather_body,
        out_type=[jax.ShapeDtypeStruct((_BC * _L, gw), gdt),
                  jax.ShapeDtypeStruct((_BC, _D), jnp.float32)],
        mesh=mesh,
        scratch_types=scratch,
    )
    return fn(uh3d, nodes3d, user_table, item_table)


def _xform_body(utab_ref, ln1_ref, out_ref):
    r = lax.dot_general(utab_ref[...], ln1_ref[...][:, :_D],
                        (((1,), (1,)), ((), ())),
                        preferred_element_type=jnp.float32)
    out_ref[...] = r


def _transform_table(user_table, ln1_w, *, rb=2000, interpret=False):
    n = user_table.shape[0]
    return pl.pallas_call(
        _xform_body,
        grid=(n // rb,),
        in_specs=[pl.BlockSpec((rb, _D), lambda i: (i, 0)),
                  pl.BlockSpec(ln1_w.shape, lambda i: (0, 0))],
        out_specs=pl.BlockSpec((rb, _D), lambda i: (i, 0)),
        out_shape=jax.ShapeDtypeStruct((n, _D), jnp.float32),
        interpret=interpret,
    )(user_table, ln1_w)


def _tc_body(gath_ref, rh_ref, wi_ref, ln1_ref, ln1b_ref, ln2_ref, ln2b_ref,
             ln3_ref, ln3b_ref, att1_ref, att1b_ref, att2_ref, att2b_ref,
             rtab_ref, out_ref, *, nb):
    f32 = jnp.float32
    dot = lambda a, b: lax.dot_general(a, b, (((1,), (1,)), ((), ())),
                                       preferred_element_type=f32)

    g = gath_ref[...]
    rh = rh_ref[...]
    wi = wi_ref[...]

    ln1 = ln1_ref[...]
    rfold = dot(rtab_ref[...], ln1[:, _D:]) + ln1b_ref[...]
    rfold8 = jnp.concatenate([rfold, jnp.zeros((2, _D), f32)], axis=0)
    oht = (rh == lax.broadcasted_iota(jnp.int32, (8, 1), 0)).astype(f32)
    rpart = lax.dot_general(oht, rfold8, (((0,), (0,)), ((), ())),
                            preferred_element_type=f32)

    f = jnp.maximum(dot(g, ln1[:, :_D]) + rpart, 0.0)

    rows = nb * _L
    att1 = att1_ref[...]
    c = dot(wi, att1[:, _D:]) + att1b_ref[...]
    hmat = dot(f, att1[:, :_D]).reshape(nb, _L, _D)
    h = jnp.maximum(hmat + c[:, None, :], 0.0).reshape(rows, _D)
    st = lax.dot_general(att2_ref[...], h, (((1,), (1,)), ((), ())),
                         preferred_element_type=f32)
    ex = jnp.exp(st - jnp.max(st))
    seg = lax.broadcasted_iota(jnp.int32, (1, rows), 1) // _L
    m = jnp.where(seg == lax.broadcasted_iota(jnp.int32, (nb, 1), 0),
                  ex, 0.0)
    denom = jnp.sum(m, axis=1, keepdims=True)
    zsum = lax.dot_general(m, f, (((1,), (0,)), ((), ())),
                           preferred_element_type=f32)
    z = zsum / denom

    z2 = jnp.maximum(dot(z, ln2_ref[...]) + ln2b_ref[...], 0.0)
    ln3 = ln3_ref[...]
    out = jnp.maximum(dot(wi, ln3[:, :_D]) + dot(z2, ln3[:, _D:])
                      + ln3b_ref[...], 0.0)
    out_ref[...] = out


def _tc_compute(gathered, rh2d, w_item, rating_table,
                ln1_w, ln1_b, ln2_w, ln2_b, ln3_w, ln3_b,
                att1_w, att1_b, att2_w, att2_b, *, nb=64, interpret=False):
    bc = w_item.shape[0]
    grid = (bc // nb,)
    row1 = lambda x: x.reshape(1, -1).astype(jnp.float32)
    full = lambda a: pl.BlockSpec(a.shape, lambda i: (0,) * a.ndim)
    in_specs = [
        pl.BlockSpec((nb * _L, gathered.shape[1]), lambda i: (i, 0)),
        pl.BlockSpec((1, nb * _L), lambda i: (0, i)),
        pl.BlockSpec((nb, _D), lambda i: (i, 0)),
        full(ln1_w), full(row1(ln1_b)), full(ln2_w), full(row1(ln2_b)),
        full(ln3_w), full(row1(ln3_b)), full(att1_w), full(row1(att1_b)),
        full(row1(att2_w)), full(row1(att2_b)), full(rating_table),
    ]
    return pl.pallas_call(
        functools.partial(_tc_body, nb=nb),
        grid=grid,
        in_specs=in_specs,
        out_specs=pl.BlockSpec((nb, _D), lambda i: (i, 0)),
        out_shape=jax.ShapeDtypeStruct((bc, _D), jnp.float32),
        interpret=interpret,
    )(gathered, rh2d, w_item, ln1_w, row1(ln1_b), ln2_w, row1(ln2_b),
      ln3_w, row1(ln3_b), att1_w, row1(att1_b), row1(att2_w), row1(att2_b),
      rating_table)


def kernel(user_hist, rating_hist, nodes, user_table, item_table,
           rating_table, ln1_w, ln1_b, ln2_w, ln2_b, ln3_w, ln3_b,
           att1_w, att1_b, att2_w, att2_b):
    rh_t = rating_hist.reshape(1, _B * _L).astype(jnp.int32)
    outs = []
    for k in range(_K):
        sl = slice(k * _BC, (k + 1) * _BC)
        gathered, w_item = _sc_gather(user_hist[sl], nodes[sl], user_table,
                                      item_table)
        outs.append(_tc_compute(
            gathered, rh_t[:, k * _BC * _L:(k + 1) * _BC * _L], w_item,
            rating_table, ln1_w, ln1_b, ln2_w, ln2_b,
            ln3_w, ln3_b, att1_w, att1_b, att2_w, att2_b))
    return jnp.concatenate(outs, axis=0)
